# Initial kernel scaffold; baseline (speedup 1.0000x reference)
import jax
import jax.numpy as jnp
from jax.experimental import pallas as pl


def kernel(node_embeddings, time_positions, node_pred, audio_len, cur_anchor_intervals, params):
    raise NotImplementedError("write your pallas kernel here")


# speedup vs baseline: 164.9642x; 164.9642x over previous
import functools

import jax
import jax.numpy as jnp
from jax import lax
from jax.experimental import pallas as pl
from jax.experimental.pallas import tpu as pltpu
from jax.experimental.pallas import tpu_sc as plsc

D = 256
NC = 5
N = 4096
BINS = (80, 60, 80)
SPLITS = (15, 40, 15)
OFFS = (0, 15, 55)
P_CAP = 512
BMAX = 40
W3_PAD = 384


SC_WORKERS = 32
NI = 70


def _sc_compact_body(time_hbm, st_hbm, en_hbm, idx_hbm, idxr_hbm, cnt_hbm,
                     time_v, st_v, en_v, idxl_v, revl_v, revo_v, cntl_v):
    wid = lax.axis_index("c") * 16 + lax.axis_index("s")
    pltpu.sync_copy(time_hbm, time_v)
    pltpu.sync_copy(st_hbm, st_v.at[pl.ds(0, NI)])
    pltpu.sync_copy(en_hbm, en_v.at[pl.ds(0, NI)])
    lanes = lax.iota(jnp.int32, 16)
    zeros16 = jnp.zeros((16,), jnp.int32)

    for k in range(3):
        b = wid + SC_WORKERS * k

        @pl.when(b < NI)
        def _():
            cbase = (b // 16) * 16
            lsel = lanes == (b - cbase)
            sv = jnp.sum(jnp.where(lsel, st_v[pl.ds(cbase, 16)], 0.0))
            ev = jnp.sum(jnp.where(lsel, en_v[pl.ds(cbase, 16)], 0.0))

            def zero_body(i, _):
                idxl_v[pl.ds(i * 16, 16)] = zeros16
                return 0

            lax.fori_loop(0, P_CAP // 16, zero_body, 0)

            def chunk(c, off):
                base = c * 16
                tv = time_v[pl.ds(base, 16)]
                m = (tv >= sv) & (tv <= ev)
                tvec = lanes + base
                safe = jnp.minimum(off, P_CAP)
                plsc.store_compressed(idxl_v.at[pl.ds(safe, 16)], tvec,
                                      mask=m)
                return off + jnp.sum(m.astype(jnp.int32))

            cntb = lax.fori_loop(0, N // 16, chunk, jnp.int32(0))
            cntl_v[...] = zeros16 + cntb
            pltpu.sync_copy(cntl_v, cnt_hbm.at[b])
            pltpu.sync_copy(idxl_v.at[pl.ds(0, P_CAP)], idx_hbm.at[b])

            def zero_rev(i, _):
                revl_v[pl.ds(i * 16, 16)] = zeros16
                return 0

            lax.fori_loop(0, (P_CAP + 16) // 16, zero_rev, 0)
            ce = jnp.minimum(cntb, P_CAP)

            def rev_chunk(c, _):
                p = c * 16
                v = idxl_v[pl.ds(p, 16)]
                revl_v[pl.ds(ce - p, 16)] = lax.rev(v, (0,))
                return 0

            lax.fori_loop(0, (ce + 15) // 16, rev_chunk, 0)

            def mv_chunk(i, _):
                revo_v[pl.ds(i * 16, 16)] = revl_v[pl.ds(i * 16 + 16, 16)]
                return 0

            lax.fori_loop(0, P_CAP // 16, mv_chunk, 0)
            pltpu.sync_copy(revo_v, idxr_hbm.at[b])


def _build_idx_sc(time1d, starts70, ends70):
    mesh = plsc.VectorSubcoreMesh(core_axis_name="c", subcore_axis_name="s")
    idx, idxr, cnt16 = pl.kernel(
        _sc_compact_body,
        mesh=mesh,
        compiler_params=pltpu.CompilerParams(needs_layout_passes=False),
        out_type=[
            jax.ShapeDtypeStruct((NI, P_CAP), jnp.int32),
            jax.ShapeDtypeStruct((NI, P_CAP), jnp.int32),
            jax.ShapeDtypeStruct((NI, 16), jnp.int32),
        ],
        scratch_types=[
            pltpu.VMEM((N,), jnp.float32),
            pltpu.VMEM((128,), jnp.float32),
            pltpu.VMEM((128,), jnp.float32),
            pltpu.VMEM((P_CAP + 16,), jnp.int32),
            pltpu.VMEM((P_CAP + 16,), jnp.int32),
            pltpu.VMEM((P_CAP,), jnp.int32),
            pltpu.VMEM((16,), jnp.int32),
        ],
    )(time1d, starts70, ends70)
    return idx, idxr, cnt16[:, 0]


def _gru_heads_body(idx_s, idxr_s, cnt_s, x_ref, np40_ref, al_ref, *rest):
    per_scale = [rest[17 * si:17 * (si + 1)] for si in range(3)]
    outs = rest[51:60]
    xtf_ref, xtb_ref = rest[60], rest[61]

    alv = al_ref[:, :]

    npv = np40_ref[:, :]
    m = jnp.max(npv, axis=1, keepdims=True)
    e = jnp.exp(npv - m)
    abn = e[:, 0:1] / jnp.sum(e, axis=1, keepdims=True)

    for si in range(3):
        (st_ref, en_ref, cv_ref,
         wihf_ref, whhf_ref, bihf_ref, bhhf_ref,
         wihb_ref, whhb_ref, bihb_ref, bhhb_ref,
         w1a_ref, w1b_ref, w2_ref, w3_ref, sw_ref, ew_ref) = per_scale[si]
        bnd_ref, cf_ref, cl_ref = outs[3 * si:3 * si + 3]
        B = SPLITS[si]
        r0 = OFFS[si]

        cnts = [cnt_s[r0 + b] for b in range(B)]
        lens = [jnp.maximum(c, 1) for c in cnts]
        lmax = functools.reduce(jnp.maximum, lens)
        lmax = jnp.minimum(lmax, P_CAP)

        cntv = cv_ref[:, :]
        lenv = jnp.maximum(cntv, 1)

        wihf = wihf_ref[:, :]
        whhf = whhf_ref[:, :]
        bihf = bihf_ref[:, :]
        bhhf = bhhf_ref[:, :]
        wihb = wihb_ref[:, :]
        whhb = whhb_ref[:, :]
        bihb = bihb_ref[:, :]
        bhhb = bhhb_ref[:, :]

        def gru_cell(xv, h, w_hh, b_hh):
            gh = jnp.dot(h, w_hh, preferred_element_type=jnp.float32) + b_hh
            r = jax.nn.sigmoid(xv[:, 0:D] + gh[:, 0:D])
            z = jax.nn.sigmoid(xv[:, D:2 * D] + gh[:, D:2 * D])
            n = jnp.tanh(xv[:, 2 * D:3 * D] + r * gh[:, 2 * D:3 * D])
            return (1.0 - z) * n + z * h

        def step(p, carry):
            hf, hb = carry
            pc = jnp.minimum(p, P_CAP - 1)
            for b in range(B):
                ifw = idx_s[r0 + b, pc]
                xtf_ref[pl.ds(b, 1), :] = x_ref[pl.ds(ifw, 1), :]
                ibw = idxr_s[r0 + b, pc]
                xtb_ref[pl.ds(b, 1), :] = x_ref[pl.ds(ibw, 1), :]
            inm = p < cntv
            xf = jnp.where(inm, xtf_ref[0:B, :], 0.0)
            xb = jnp.where(inm, xtb_ref[0:B, :], 0.0)
            gif = jnp.dot(xf, wihf, preferred_element_type=jnp.float32) + bihf
            gib = jnp.dot(xb, wihb, preferred_element_type=jnp.float32) + bihb
            hf2 = gru_cell(gif, hf, whhf, bhhf)
            hb2 = gru_cell(gib, hb, whhb, bhhb)
            upd = p < lenv
            return jnp.where(upd, hf2, hf), jnp.where(upd, hb2, hb)

        h0 = jnp.zeros((B, D), jnp.float32)
        hf, hb = lax.fori_loop(0, lmax, step, (h0, h0))

        startsv = st_ref[:, :]
        endsv = en_ref[:, :]
        mid = (startsv + endsv) * 0.5 / alv
        wid = (endsv - startsv) / alv

        local = jnp.concatenate([hf, hb], axis=1)
        h1 = jnp.dot(local, w1a_ref[:, :], preferred_element_type=jnp.float32)
        h1 = h1 + abn[0:B, :] * w1b_ref[0:1, :]
        h1 = h1 + mid * w1b_ref[1:2, :]
        h1 = h1 + wid * w1b_ref[2:3, :]
        h1 = jnp.maximum(h1, 0.0)
        h2 = jnp.maximum(
            jnp.dot(h1, w2_ref[:, :], preferred_element_type=jnp.float32), 0.0)
        outp = jnp.dot(h2, w3_ref[:, :], preferred_element_type=jnp.float32)

        bsz = BINS[si]

        def bin_expect(logits, w):
            mm = jnp.max(logits, axis=1, keepdims=True)
            ee = jnp.exp(logits - mm)
            pp = ee / jnp.sum(ee, axis=1, keepdims=True)
            return jnp.sum(pp * w, axis=1, keepdims=True)

        so = bin_expect(outp[:, 0:bsz], sw_ref[0:1, 0:bsz])
        eo = bin_expect(outp[:, 128:128 + bsz], ew_ref[0:1, 0:bsz])
        ns = jnp.minimum(jnp.maximum(startsv + so, 0.0), alv)
        ne = jnp.minimum(jnp.maximum(endsv + eo, 0.0), alv)
        bnd_ref[:, 0:1] = ns
        bnd_ref[:, 1:2] = ne
        cf_ref[:, :] = outp[:, 256:257]
        cl_ref[:, :] = outp[:, 257:261]


def kernel(node_embeddings, time_positions, node_pred, audio_len,
           cur_anchor_intervals, params):
    al = audio_len.reshape(-1)[0]
    time1d = time_positions.reshape(-1) * al
    starts70 = cur_anchor_intervals[:, 0]
    ends70 = cur_anchor_intervals[:, 1]

    idx, idxr, cnt = _build_idx_sc(time1d, starts70, ends70)

    np40 = node_pred[:BMAX, :]
    al_arr = audio_len.reshape(1, 1)

    args = [idx, idxr, cnt, node_embeddings, np40, al_arr]
    for si in range(3):
        r0, B = OFFS[si], SPLITS[si]
        args.append(starts70[r0:r0 + B].reshape(B, 1))
        args.append(ends70[r0:r0 + B].reshape(B, 1))
        args.append(cnt[r0:r0 + B].reshape(B, 1))
        for d in ("f", "b"):
            args.append(params["w_ih_" + d][si].T)
            args.append(params["w_hh_" + d][si].T)
            args.append(params["b_ih_" + d][si].reshape(1, 3 * D))
            args.append(params["b_hh_" + d][si].reshape(1, 3 * D))
        W1 = params["W1"][si]
        args.append(W1[:, :2 * D].T)
        args.append(W1[:, 2 * D:].T)
        args.append(params["W2"][si].T)
        W3 = params["W3"][si]
        bsz = BINS[si]
        w3p = jnp.zeros((D, W3_PAD), jnp.float32)
        w3p = w3p.at[:, 0:bsz].set(W3[0:bsz].T)
        w3p = w3p.at[:, 128:128 + bsz].set(W3[bsz:2 * bsz].T)
        w3p = w3p.at[:, 256:257].set(W3[2 * bsz:2 * bsz + 1].T)
        w3p = w3p.at[:, 257:261].set(W3[2 * bsz + 1:2 * bsz + 5].T)
        args.append(w3p)
        swp = jnp.zeros((1, 128), jnp.float32)
        args.append(swp.at[0, :bsz].set(params["start_w"][si]))
        args.append(swp.at[0, :bsz].set(params["end_w"][si]))

    smem_spec = pl.BlockSpec(memory_space=pltpu.SMEM)
    vmem_spec = pl.BlockSpec(memory_space=pltpu.VMEM)
    in_specs = [smem_spec] * 3 + [vmem_spec] * (len(args) - 3)

    out_shapes = []
    for si in range(3):
        B = SPLITS[si]
        out_shapes.append(jax.ShapeDtypeStruct((B, 2), jnp.float32))
        out_shapes.append(jax.ShapeDtypeStruct((B, 1), jnp.float32))
        out_shapes.append(jax.ShapeDtypeStruct((B, NC - 1), jnp.float32))

    outs = pl.pallas_call(
        _gru_heads_body,
        in_specs=in_specs,
        out_specs=[vmem_spec] * 9,
        out_shape=out_shapes,
        scratch_shapes=[pltpu.VMEM((BMAX, D), jnp.float32)] * 2,
    )(*args)

    bounds = jnp.concatenate([outs[0], outs[3], outs[6]], axis=0)
    confs = jnp.concatenate([outs[1], outs[4], outs[7]], axis=0).reshape(-1)
    clss = jnp.concatenate([outs[2], outs[5], outs[8]], axis=0)
    return bounds, confs, clss


# speedup vs baseline: 201.8609x; 1.2237x over previous
import functools

import jax
import jax.numpy as jnp
from jax import lax
from jax.experimental import pallas as pl
from jax.experimental.pallas import tpu as pltpu
from jax.experimental.pallas import tpu_sc as plsc

D = 256
NC = 5
N = 4096
BINS = (80, 60, 80)
SPLITS = (15, 40, 15)
OFFS = (0, 15, 55)
P_CAP = 512
BMAX = 40
W3_PAD = 384


SC_WORKERS = 32
NI = 70


def _sc_compact_body(time_hbm, st_hbm, en_hbm, idx_hbm, idxr_hbm, cnt_hbm,
                     time_v, st_v, en_v, idxl_v, revl_v, revo_v, cntl_v):
    wid = lax.axis_index("c") * 16 + lax.axis_index("s")
    pltpu.sync_copy(time_hbm, time_v)
    pltpu.sync_copy(st_hbm, st_v.at[pl.ds(0, NI)])
    pltpu.sync_copy(en_hbm, en_v.at[pl.ds(0, NI)])
    lanes = lax.iota(jnp.int32, 16)
    zeros16 = jnp.zeros((16,), jnp.int32)

    for k in range(3):
        b = wid + SC_WORKERS * k

        @pl.when(b < NI)
        def _():
            cbase = (b // 16) * 16
            lsel = lanes == (b - cbase)
            sv = jnp.sum(jnp.where(lsel, st_v[pl.ds(cbase, 16)], 0.0))
            ev = jnp.sum(jnp.where(lsel, en_v[pl.ds(cbase, 16)], 0.0))

            def zero_body(i, _):
                idxl_v[pl.ds(i * 16, 16)] = zeros16
                return 0

            lax.fori_loop(0, P_CAP // 16, zero_body, 0)

            def chunk(c, off):
                base = c * 16
                tv = time_v[pl.ds(base, 16)]
                m = (tv >= sv) & (tv <= ev)
                tvec = lanes + base
                safe = jnp.minimum(off, P_CAP)
                plsc.store_compressed(idxl_v.at[pl.ds(safe, 16)], tvec,
                                      mask=m)
                return off + jnp.sum(m.astype(jnp.int32))

            cntb = lax.fori_loop(0, N // 16, chunk, jnp.int32(0))
            cntl_v[...] = zeros16 + cntb
            pltpu.sync_copy(cntl_v, cnt_hbm.at[b])
            pltpu.sync_copy(idxl_v.at[pl.ds(0, P_CAP)], idx_hbm.at[b])

            def zero_rev(i, _):
                revl_v[pl.ds(i * 16, 16)] = zeros16
                return 0

            lax.fori_loop(0, (P_CAP + 16) // 16, zero_rev, 0)
            ce = jnp.minimum(cntb, P_CAP)

            def rev_chunk(c, _):
                p = c * 16
                v = idxl_v[pl.ds(p, 16)]
                revl_v[pl.ds(ce - p, 16)] = lax.rev(v, (0,))
                return 0

            lax.fori_loop(0, (ce + 15) // 16, rev_chunk, 0)

            def mv_chunk(i, _):
                revo_v[pl.ds(i * 16, 16)] = revl_v[pl.ds(i * 16 + 16, 16)]
                return 0

            lax.fori_loop(0, P_CAP // 16, mv_chunk, 0)
            pltpu.sync_copy(revo_v, idxr_hbm.at[b])


def _build_idx_sc(time1d, starts70, ends70):
    mesh = plsc.VectorSubcoreMesh(core_axis_name="c", subcore_axis_name="s")
    idx, idxr, cnt16 = pl.kernel(
        _sc_compact_body,
        mesh=mesh,
        compiler_params=pltpu.CompilerParams(needs_layout_passes=False),
        out_type=[
            jax.ShapeDtypeStruct((NI, P_CAP), jnp.int32),
            jax.ShapeDtypeStruct((NI, P_CAP), jnp.int32),
            jax.ShapeDtypeStruct((NI, 16), jnp.int32),
        ],
        scratch_types=[
            pltpu.VMEM((N,), jnp.float32),
            pltpu.VMEM((128,), jnp.float32),
            pltpu.VMEM((128,), jnp.float32),
            pltpu.VMEM((P_CAP + 16,), jnp.int32),
            pltpu.VMEM((P_CAP + 16,), jnp.int32),
            pltpu.VMEM((P_CAP,), jnp.int32),
            pltpu.VMEM((16,), jnp.int32),
        ],
    )(time1d, starts70, ends70)
    return idx, idxr, cnt16[:, 0]


def _gru_cell(xv, h, w_hh, b_hh):
    gh = jnp.dot(h, w_hh, preferred_element_type=jnp.float32) + b_hh
    r = jax.nn.sigmoid(xv[:, 0:D] + gh[:, 0:D])
    z = jax.nn.sigmoid(xv[:, D:2 * D] + gh[:, D:2 * D])
    n = jnp.tanh(xv[:, 2 * D:3 * D] + r * gh[:, 2 * D:3 * D])
    return (1.0 - z) * n + z * h


def _gru_heads_body(idx_s, idxr_s, cnt_s, x_ref, np40_ref, al_ref, *rest):
    per_scale = [rest[17 * si:17 * (si + 1)] for si in range(3)]
    outs = rest[51:60]
    scratch = rest[60:66]

    alv = al_ref[:, :]

    npv = np40_ref[:, :]
    m = jnp.max(npv, axis=1, keepdims=True)
    e = jnp.exp(npv - m)
    abn = e[:, 0:1] / jnp.sum(e, axis=1, keepdims=True)

    lens_all = [jnp.maximum(cnt_s[i], 1) for i in range(NI)]
    lmax = functools.reduce(jnp.maximum, lens_all)
    lmax = jnp.minimum(lmax, P_CAP)

    prep = []
    for si in range(3):
        (st_ref, en_ref, cv_ref,
         wihf_ref, whhf_ref, bihf_ref, bhhf_ref,
         wihb_ref, whhb_ref, bihb_ref, bhhb_ref,
         w1a_ref, w1b_ref, w2_ref, w3_ref, sw_ref, ew_ref) = per_scale[si]
        cntv = cv_ref[:, :]
        prep.append(dict(
            cntv=cntv, lenv=jnp.maximum(cntv, 1),
            wihf=wihf_ref[:, :], whhf=whhf_ref[:, :],
            bihf=bihf_ref[:, :], bhhf=bhhf_ref[:, :],
            wihb=wihb_ref[:, :], whhb=whhb_ref[:, :],
            bihb=bihb_ref[:, :], bhhb=bhhb_ref[:, :],
        ))

    def step(p, carry):
        pc = jnp.minimum(p, P_CAP - 1)
        new = []
        for si in range(3):
            hf, hb = carry[2 * si], carry[2 * si + 1]
            B, r0 = SPLITS[si], OFFS[si]
            xtf_ref, xtb_ref = scratch[2 * si], scratch[2 * si + 1]
            pr = prep[si]
            for b in range(B):
                ifw = idx_s[r0 + b, pc]
                xtf_ref[pl.ds(b, 1), :] = x_ref[pl.ds(ifw, 1), :]
                ibw = idxr_s[r0 + b, pc]
                xtb_ref[pl.ds(b, 1), :] = x_ref[pl.ds(ibw, 1), :]
            inm = p < pr["cntv"]
            xf = jnp.where(inm, xtf_ref[:, :], 0.0)
            xb = jnp.where(inm, xtb_ref[:, :], 0.0)
            gif = jnp.dot(xf, pr["wihf"],
                          preferred_element_type=jnp.float32) + pr["bihf"]
            gib = jnp.dot(xb, pr["wihb"],
                          preferred_element_type=jnp.float32) + pr["bihb"]
            hf2 = _gru_cell(gif, hf, pr["whhf"], pr["bhhf"])
            hb2 = _gru_cell(gib, hb, pr["whhb"], pr["bhhb"])
            upd = p < pr["lenv"]
            new.append(jnp.where(upd, hf2, hf))
            new.append(jnp.where(upd, hb2, hb))
        return tuple(new)

    h0s = tuple(jnp.zeros((SPLITS[si // 2], D), jnp.float32)
                for si in range(6))
    hfin = lax.fori_loop(0, lmax, step, h0s)

    for si in range(3):
        (st_ref, en_ref, cv_ref,
         wihf_ref, whhf_ref, bihf_ref, bhhf_ref,
         wihb_ref, whhb_ref, bihb_ref, bhhb_ref,
         w1a_ref, w1b_ref, w2_ref, w3_ref, sw_ref, ew_ref) = per_scale[si]
        bnd_ref, cf_ref, cl_ref = outs[3 * si:3 * si + 3]
        B = SPLITS[si]
        hf, hb = hfin[2 * si], hfin[2 * si + 1]

        startsv = st_ref[:, :]
        endsv = en_ref[:, :]
        mid = (startsv + endsv) * 0.5 / alv
        wid = (endsv - startsv) / alv

        local = jnp.concatenate([hf, hb], axis=1)
        h1 = jnp.dot(local, w1a_ref[:, :], preferred_element_type=jnp.float32)
        h1 = h1 + abn[0:B, :] * w1b_ref[0:1, :]
        h1 = h1 + mid * w1b_ref[1:2, :]
        h1 = h1 + wid * w1b_ref[2:3, :]
        h1 = jnp.maximum(h1, 0.0)
        h2 = jnp.maximum(
            jnp.dot(h1, w2_ref[:, :], preferred_element_type=jnp.float32), 0.0)
        outp = jnp.dot(h2, w3_ref[:, :], preferred_element_type=jnp.float32)

        bsz = BINS[si]

        def bin_expect(logits, w):
            mm = jnp.max(logits, axis=1, keepdims=True)
            ee = jnp.exp(logits - mm)
            pp = ee / jnp.sum(ee, axis=1, keepdims=True)
            return jnp.sum(pp * w, axis=1, keepdims=True)

        so = bin_expect(outp[:, 0:bsz], sw_ref[0:1, 0:bsz])
        eo = bin_expect(outp[:, 128:128 + bsz], ew_ref[0:1, 0:bsz])
        ns = jnp.minimum(jnp.maximum(startsv + so, 0.0), alv)
        ne = jnp.minimum(jnp.maximum(endsv + eo, 0.0), alv)
        bnd_ref[:, 0:1] = ns
        bnd_ref[:, 1:2] = ne
        cf_ref[:, :] = outp[:, 256:257]
        cl_ref[:, :] = outp[:, 257:261]


def kernel(node_embeddings, time_positions, node_pred, audio_len,
           cur_anchor_intervals, params):
    al = audio_len.reshape(-1)[0]
    time1d = time_positions.reshape(-1) * al
    starts70 = cur_anchor_intervals[:, 0]
    ends70 = cur_anchor_intervals[:, 1]

    idx, idxr, cnt = _build_idx_sc(time1d, starts70, ends70)

    np40 = node_pred[:BMAX, :]
    al_arr = audio_len.reshape(1, 1)

    args = [idx, idxr, cnt, node_embeddings, np40, al_arr]
    for si in range(3):
        r0, B = OFFS[si], SPLITS[si]
        args.append(starts70[r0:r0 + B].reshape(B, 1))
        args.append(ends70[r0:r0 + B].reshape(B, 1))
        args.append(cnt[r0:r0 + B].reshape(B, 1))
        for d in ("f", "b"):
            args.append(params["w_ih_" + d][si].T)
            args.append(params["w_hh_" + d][si].T)
            args.append(params["b_ih_" + d][si].reshape(1, 3 * D))
            args.append(params["b_hh_" + d][si].reshape(1, 3 * D))
        W1 = params["W1"][si]
        args.append(W1[:, :2 * D].T)
        args.append(W1[:, 2 * D:].T)
        args.append(params["W2"][si].T)
        W3 = params["W3"][si]
        bsz = BINS[si]
        w3p = jnp.zeros((D, W3_PAD), jnp.float32)
        w3p = w3p.at[:, 0:bsz].set(W3[0:bsz].T)
        w3p = w3p.at[:, 128:128 + bsz].set(W3[bsz:2 * bsz].T)
        w3p = w3p.at[:, 256:257].set(W3[2 * bsz:2 * bsz + 1].T)
        w3p = w3p.at[:, 257:261].set(W3[2 * bsz + 1:2 * bsz + 5].T)
        args.append(w3p)
        swp = jnp.zeros((1, 128), jnp.float32)
        args.append(swp.at[0, :bsz].set(params["start_w"][si]))
        args.append(swp.at[0, :bsz].set(params["end_w"][si]))

    smem_spec = pl.BlockSpec(memory_space=pltpu.SMEM)
    vmem_spec = pl.BlockSpec(memory_space=pltpu.VMEM)
    in_specs = [smem_spec] * 3 + [vmem_spec] * (len(args) - 3)

    out_shapes = []
    for si in range(3):
        B = SPLITS[si]
        out_shapes.append(jax.ShapeDtypeStruct((B, 2), jnp.float32))
        out_shapes.append(jax.ShapeDtypeStruct((B, 1), jnp.float32))
        out_shapes.append(jax.ShapeDtypeStruct((B, NC - 1), jnp.float32))

    outs = pl.pallas_call(
        _gru_heads_body,
        in_specs=in_specs,
        out_specs=[vmem_spec] * 9,
        out_shape=out_shapes,
        scratch_shapes=[pltpu.VMEM((SPLITS[i // 2], D), jnp.float32)
                        for i in range(6)],
    )(*args)

    bounds = jnp.concatenate([outs[0], outs[3], outs[6]], axis=0)
    confs = jnp.concatenate([outs[1], outs[4], outs[7]], axis=0).reshape(-1)
    clss = jnp.concatenate([outs[2], outs[5], outs[8]], axis=0)
    return bounds, confs, clss


# speedup vs baseline: 213.6033x; 1.0582x over previous
import functools

import jax
import jax.numpy as jnp
from jax import lax
from jax.experimental import pallas as pl
from jax.experimental.pallas import tpu as pltpu
from jax.experimental.pallas import tpu_sc as plsc

D = 256
NC = 5
N = 4096
BINS = (80, 60, 80)
SPLITS = (15, 40, 15)
OFFS = (0, 15, 55)
P_CAP = 512
BMAX = 40
W3_PAD = 384


SC_WORKERS = 32
NI = 70


def _sc_compact_body(time_hbm, st_hbm, en_hbm, idx_hbm, idxr_hbm, cnt_hbm,
                     time_v, st_v, en_v, idxl_v, revl_v, revo_v, cntl_v):
    wid = lax.axis_index("c") * 16 + lax.axis_index("s")
    pltpu.sync_copy(time_hbm, time_v)
    pltpu.sync_copy(st_hbm, st_v.at[pl.ds(0, NI)])
    pltpu.sync_copy(en_hbm, en_v.at[pl.ds(0, NI)])
    lanes = lax.iota(jnp.int32, 16)
    zeros16 = jnp.zeros((16,), jnp.int32)

    for k in range(3):
        b = wid + SC_WORKERS * k

        @pl.when(b < NI)
        def _():
            cbase = (b // 16) * 16
            lsel = lanes == (b - cbase)
            sv = jnp.sum(jnp.where(lsel, st_v[pl.ds(cbase, 16)], 0.0))
            ev = jnp.sum(jnp.where(lsel, en_v[pl.ds(cbase, 16)], 0.0))

            def zero_body(i, _):
                idxl_v[pl.ds(i * 16, 16)] = zeros16
                return 0

            lax.fori_loop(0, P_CAP // 16, zero_body, 0)

            def chunk(c, off):
                base = c * 16
                tv = time_v[pl.ds(base, 16)]
                m = (tv >= sv) & (tv <= ev)
                tvec = lanes + base
                safe = jnp.minimum(off, P_CAP)
                plsc.store_compressed(idxl_v.at[pl.ds(safe, 16)], tvec,
                                      mask=m)
                return off + jnp.sum(m.astype(jnp.int32))

            cntb = lax.fori_loop(0, N // 16, chunk, jnp.int32(0))
            cntl_v[...] = zeros16 + cntb
            pltpu.sync_copy(cntl_v, cnt_hbm.at[b])
            pltpu.sync_copy(idxl_v.at[pl.ds(0, P_CAP)], idx_hbm.at[b])

            def zero_rev(i, _):
                revl_v[pl.ds(i * 16, 16)] = zeros16
                return 0

            lax.fori_loop(0, (P_CAP + 16) // 16, zero_rev, 0)
            ce = jnp.minimum(cntb, P_CAP)

            def rev_chunk(c, _):
                p = c * 16
                v = idxl_v[pl.ds(p, 16)]
                revl_v[pl.ds(ce - p, 16)] = lax.rev(v, (0,))
                return 0

            lax.fori_loop(0, (ce + 15) // 16, rev_chunk, 0)

            def mv_chunk(i, _):
                revo_v[pl.ds(i * 16, 16)] = revl_v[pl.ds(i * 16 + 16, 16)]
                return 0

            lax.fori_loop(0, P_CAP // 16, mv_chunk, 0)
            pltpu.sync_copy(revo_v, idxr_hbm.at[b])


def _build_idx_sc(time1d, starts70, ends70):
    mesh = plsc.VectorSubcoreMesh(core_axis_name="c", subcore_axis_name="s")
    idx, idxr, cnt16 = pl.kernel(
        _sc_compact_body,
        mesh=mesh,
        compiler_params=pltpu.CompilerParams(needs_layout_passes=False),
        out_type=[
            jax.ShapeDtypeStruct((NI, P_CAP), jnp.int32),
            jax.ShapeDtypeStruct((NI, P_CAP), jnp.int32),
            jax.ShapeDtypeStruct((NI, 16), jnp.int32),
        ],
        scratch_types=[
            pltpu.VMEM((N,), jnp.float32),
            pltpu.VMEM((128,), jnp.float32),
            pltpu.VMEM((128,), jnp.float32),
            pltpu.VMEM((P_CAP + 16,), jnp.int32),
            pltpu.VMEM((P_CAP + 16,), jnp.int32),
            pltpu.VMEM((P_CAP,), jnp.int32),
            pltpu.VMEM((16,), jnp.int32),
        ],
    )(time1d, starts70, ends70)
    return idx, idxr, cnt16[:, 0]


def _gru_cell(xv, h, w_hh, b_hh):
    gh = jnp.dot(h, w_hh, preferred_element_type=jnp.float32) + b_hh
    r = jax.nn.sigmoid(xv[:, 0:D] + gh[:, 0:D])
    z = jax.nn.sigmoid(xv[:, D:2 * D] + gh[:, D:2 * D])
    n = jnp.tanh(xv[:, 2 * D:3 * D] + r * gh[:, 2 * D:3 * D])
    return (1.0 - z) * n + z * h


def _gru_heads_body(idx_s, idxr_s, cnt_s, x_ref, np40_ref, al_ref, *rest):
    per_scale = [rest[17 * si:17 * (si + 1)] for si in range(3)]
    outs = rest[51:60]
    scratch = rest[60:66]

    alv = al_ref[:, :]

    npv = np40_ref[:, :]
    m = jnp.max(npv, axis=1, keepdims=True)
    e = jnp.exp(npv - m)
    abn = e[:, 0:1] / jnp.sum(e, axis=1, keepdims=True)

    lens_all = [jnp.maximum(cnt_s[i], 1) for i in range(NI)]
    lmax = functools.reduce(jnp.maximum, lens_all)
    lmax = jnp.minimum(lmax, P_CAP)

    prep = []
    for si in range(3):
        (st_ref, en_ref, cv_ref,
         wihf_ref, whhf_ref, bihf_ref, bhhf_ref,
         wihb_ref, whhb_ref, bihb_ref, bhhb_ref,
         w1a_ref, w1b_ref, w2_ref, w3_ref, sw_ref, ew_ref) = per_scale[si]
        cntv = cv_ref[:, :]
        prep.append(dict(
            cntv=cntv, lenv=jnp.maximum(cntv, 1),
            wihf=wihf_ref[:, :], whhf=whhf_ref[:, :],
            bihf=bihf_ref[:, :], bhhf=bhhf_ref[:, :],
            wihb=wihb_ref[:, :], whhb=whhb_ref[:, :],
            bihb=bihb_ref[:, :], bhhb=bhhb_ref[:, :],
        ))

    def substep(p, carry):
        pc = jnp.minimum(p, P_CAP - 1)
        new = []
        for si in range(3):
            hf, hb = carry[2 * si], carry[2 * si + 1]
            B, r0 = SPLITS[si], OFFS[si]
            xtf_ref, xtb_ref = scratch[2 * si], scratch[2 * si + 1]
            pr = prep[si]
            for b in range(B):
                ifw = idx_s[r0 + b, pc]
                xtf_ref[pl.ds(b, 1), :] = x_ref[pl.ds(ifw, 1), :]
                ibw = idxr_s[r0 + b, pc]
                xtb_ref[pl.ds(b, 1), :] = x_ref[pl.ds(ibw, 1), :]
            inm = p < pr["cntv"]
            xf = jnp.where(inm, xtf_ref[:, :], 0.0)
            xb = jnp.where(inm, xtb_ref[:, :], 0.0)
            gif = jnp.dot(xf, pr["wihf"],
                          preferred_element_type=jnp.float32) + pr["bihf"]
            gib = jnp.dot(xb, pr["wihb"],
                          preferred_element_type=jnp.float32) + pr["bihb"]
            hf2 = _gru_cell(gif, hf, pr["whhf"], pr["bhhf"])
            hb2 = _gru_cell(gib, hb, pr["whhb"], pr["bhhb"])
            upd = p < pr["lenv"]
            new.append(jnp.where(upd, hf2, hf))
            new.append(jnp.where(upd, hb2, hb))
        return tuple(new)

    def step(q, carry):
        return substep(2 * q + 1, substep(2 * q, carry))

    h0s = tuple(jnp.zeros((SPLITS[si // 2], D), jnp.float32)
                for si in range(6))
    hfin = lax.fori_loop(0, (lmax + 1) // 2, step, h0s)

    for si in range(3):
        (st_ref, en_ref, cv_ref,
         wihf_ref, whhf_ref, bihf_ref, bhhf_ref,
         wihb_ref, whhb_ref, bihb_ref, bhhb_ref,
         w1a_ref, w1b_ref, w2_ref, w3_ref, sw_ref, ew_ref) = per_scale[si]
        bnd_ref, cf_ref, cl_ref = outs[3 * si:3 * si + 3]
        B = SPLITS[si]
        hf, hb = hfin[2 * si], hfin[2 * si + 1]

        startsv = st_ref[:, :]
        endsv = en_ref[:, :]
        mid = (startsv + endsv) * 0.5 / alv
        wid = (endsv - startsv) / alv

        local = jnp.concatenate([hf, hb], axis=1)
        h1 = jnp.dot(local, w1a_ref[:, :], preferred_element_type=jnp.float32)
        h1 = h1 + abn[0:B, :] * w1b_ref[0:1, :]
        h1 = h1 + mid * w1b_ref[1:2, :]
        h1 = h1 + wid * w1b_ref[2:3, :]
        h1 = jnp.maximum(h1, 0.0)
        h2 = jnp.maximum(
            jnp.dot(h1, w2_ref[:, :], preferred_element_type=jnp.float32), 0.0)
        outp = jnp.dot(h2, w3_ref[:, :], preferred_element_type=jnp.float32)

        bsz = BINS[si]

        def bin_expect(logits, w):
            mm = jnp.max(logits, axis=1, keepdims=True)
            ee = jnp.exp(logits - mm)
            pp = ee / jnp.sum(ee, axis=1, keepdims=True)
            return jnp.sum(pp * w, axis=1, keepdims=True)

        so = bin_expect(outp[:, 0:bsz], sw_ref[0:1, 0:bsz])
        eo = bin_expect(outp[:, 128:128 + bsz], ew_ref[0:1, 0:bsz])
        ns = jnp.minimum(jnp.maximum(startsv + so, 0.0), alv)
        ne = jnp.minimum(jnp.maximum(endsv + eo, 0.0), alv)
        bnd_ref[:, 0:1] = ns
        bnd_ref[:, 1:2] = ne
        cf_ref[:, :] = outp[:, 256:257]
        cl_ref[:, :] = outp[:, 257:261]


def kernel(node_embeddings, time_positions, node_pred, audio_len,
           cur_anchor_intervals, params):
    al = audio_len.reshape(-1)[0]
    time1d = time_positions.reshape(-1) * al
    starts70 = cur_anchor_intervals[:, 0]
    ends70 = cur_anchor_intervals[:, 1]

    idx, idxr, cnt = _build_idx_sc(time1d, starts70, ends70)

    np40 = node_pred[:BMAX, :]
    al_arr = audio_len.reshape(1, 1)

    args = [idx, idxr, cnt, node_embeddings, np40, al_arr]
    for si in range(3):
        r0, B = OFFS[si], SPLITS[si]
        args.append(starts70[r0:r0 + B].reshape(B, 1))
        args.append(ends70[r0:r0 + B].reshape(B, 1))
        args.append(cnt[r0:r0 + B].reshape(B, 1))
        for d in ("f", "b"):
            args.append(params["w_ih_" + d][si].T)
            args.append(params["w_hh_" + d][si].T)
            args.append(params["b_ih_" + d][si].reshape(1, 3 * D))
            args.append(params["b_hh_" + d][si].reshape(1, 3 * D))
        W1 = params["W1"][si]
        args.append(W1[:, :2 * D].T)
        args.append(W1[:, 2 * D:].T)
        args.append(params["W2"][si].T)
        W3 = params["W3"][si]
        bsz = BINS[si]
        w3p = jnp.zeros((D, W3_PAD), jnp.float32)
        w3p = w3p.at[:, 0:bsz].set(W3[0:bsz].T)
        w3p = w3p.at[:, 128:128 + bsz].set(W3[bsz:2 * bsz].T)
        w3p = w3p.at[:, 256:257].set(W3[2 * bsz:2 * bsz + 1].T)
        w3p = w3p.at[:, 257:261].set(W3[2 * bsz + 1:2 * bsz + 5].T)
        args.append(w3p)
        swp = jnp.zeros((1, 128), jnp.float32)
        args.append(swp.at[0, :bsz].set(params["start_w"][si]))
        args.append(swp.at[0, :bsz].set(params["end_w"][si]))

    smem_spec = pl.BlockSpec(memory_space=pltpu.SMEM)
    vmem_spec = pl.BlockSpec(memory_space=pltpu.VMEM)
    in_specs = [smem_spec] * 3 + [vmem_spec] * (len(args) - 3)

    out_shapes = []
    for si in range(3):
        B = SPLITS[si]
        out_shapes.append(jax.ShapeDtypeStruct((B, 2), jnp.float32))
        out_shapes.append(jax.ShapeDtypeStruct((B, 1), jnp.float32))
        out_shapes.append(jax.ShapeDtypeStruct((B, NC - 1), jnp.float32))

    outs = pl.pallas_call(
        _gru_heads_body,
        in_specs=in_specs,
        out_specs=[vmem_spec] * 9,
        out_shape=out_shapes,
        scratch_shapes=[pltpu.VMEM((SPLITS[i // 2], D), jnp.float32)
                        for i in range(6)],
    )(*args)

    bounds = jnp.concatenate([outs[0], outs[3], outs[6]], axis=0)
    confs = jnp.concatenate([outs[1], outs[4], outs[7]], axis=0).reshape(-1)
    clss = jnp.concatenate([outs[2], outs[5], outs[8]], axis=0)
    return bounds, confs, clss
